# Initial kernel scaffold; baseline (speedup 1.0000x reference)
import jax
import jax.numpy as jnp
from jax.experimental import pallas as pl


def kernel(selected_stem, one_hot_vector, stem_data):
    raise NotImplementedError("write your pallas kernel here")


# speedup vs baseline: 6.6288x; 6.6288x over previous
import jax
import jax.numpy as jnp
from jax.experimental import pallas as pl
from jax.experimental.pallas import tpu as pltpu

SHIFT = 8192


def _shift_body(onehot_ref, stem_ref, sel_ref, wav_ref):
    b = pl.program_id(0)
    row = onehot_ref[pl.ds(b, 1), :]
    idx = jnp.argmax(row)
    blk = stem_ref[...]
    wav_ref[...] = blk
    sel_ref[...] = stem_ref[0, pl.ds(idx, 1), :, :]


def kernel(selected_stem, one_hot_vector, stem_data):
    B, S, C, T = stem_data.shape
    length = T - SHIFT
    BT = length // 8
    grid = (B, length // BT)

    sel, wav = pl.pallas_call(
        _shift_body,
        grid=grid,
        in_specs=[
            pl.BlockSpec((B, S), lambda b, t: (0, 0)),
            pl.BlockSpec((1, S, C, BT), lambda b, t: (b, 0, 0, t)),
        ],
        out_specs=[
            pl.BlockSpec((1, C, BT), lambda b, t: (b, 0, t)),
            pl.BlockSpec((1, S, C, BT), lambda b, t: (b, 0, 0, t)),
        ],
        out_shape=[
            jax.ShapeDtypeStruct((B, C, length), stem_data.dtype),
            jax.ShapeDtypeStruct((B, S, C, length), stem_data.dtype),
        ],
        compiler_params=pltpu.CompilerParams(
            dimension_semantics=("parallel", "parallel"),
        ),
    )(one_hot_vector, stem_data)

    return (sel, one_hot_vector, wav)


# speedup vs baseline: 9.7786x; 1.4752x over previous
import jax
import jax.numpy as jnp
from jax.experimental import pallas as pl
from jax.experimental.pallas import tpu as pltpu

SHIFT = 8192


def _shift_body(onehot_ref, stem_ref, sel_ref, wav_ref):
    b = pl.program_id(0)
    row = onehot_ref[pl.ds(b, 1), :]
    idx = jnp.argmax(row)
    blk = stem_ref[...]
    wav_ref[...] = blk
    sel_ref[...] = stem_ref[0, pl.ds(idx, 1), :, :]


def kernel(selected_stem, one_hot_vector, stem_data):
    B, S, C, T = stem_data.shape
    length = T - SHIFT
    BT = length // 2
    grid = (B, length // BT)

    sel, wav = pl.pallas_call(
        _shift_body,
        grid=grid,
        in_specs=[
            pl.BlockSpec((B, S), lambda b, t: (0, 0)),
            pl.BlockSpec((1, S, C, BT), lambda b, t: (b, 0, 0, t)),
        ],
        out_specs=[
            pl.BlockSpec((1, C, BT), lambda b, t: (b, 0, t)),
            pl.BlockSpec((1, S, C, BT), lambda b, t: (b, 0, 0, t)),
        ],
        out_shape=[
            jax.ShapeDtypeStruct((B, C, length), stem_data.dtype),
            jax.ShapeDtypeStruct((B, S, C, length), stem_data.dtype),
        ],
        compiler_params=pltpu.CompilerParams(
            dimension_semantics=("parallel", "parallel"),
        ),
    )(one_hot_vector, stem_data)

    return (sel, one_hot_vector, wav)


# speedup vs baseline: 10.0861x; 1.0314x over previous
import jax
import jax.numpy as jnp
from jax.experimental import pallas as pl
from jax.experimental.pallas import tpu as pltpu

SHIFT = 8192


def _shift_body(onehot_ref, stem_ref, sel_ref, wav_ref):
    b = pl.program_id(0)
    row = onehot_ref[pl.ds(b, 1), :]
    idx = jnp.argmax(row)
    blk = stem_ref[...]
    wav_ref[...] = blk
    sel_ref[...] = stem_ref[0, pl.ds(idx, 1), :, :]


def kernel(selected_stem, one_hot_vector, stem_data):
    B, S, C, T = stem_data.shape
    length = T - SHIFT
    BT = length
    grid = (B, length // BT)

    sel, wav = pl.pallas_call(
        _shift_body,
        grid=grid,
        in_specs=[
            pl.BlockSpec((B, S), lambda b, t: (0, 0)),
            pl.BlockSpec((1, S, C, BT), lambda b, t: (b, 0, 0, t)),
        ],
        out_specs=[
            pl.BlockSpec((1, C, BT), lambda b, t: (b, 0, t)),
            pl.BlockSpec((1, S, C, BT), lambda b, t: (b, 0, 0, t)),
        ],
        out_shape=[
            jax.ShapeDtypeStruct((B, C, length), stem_data.dtype),
            jax.ShapeDtypeStruct((B, S, C, length), stem_data.dtype),
        ],
        compiler_params=pltpu.CompilerParams(
            dimension_semantics=("parallel", "parallel"),
        ),
    )(one_hot_vector, stem_data)

    return (sel, one_hot_vector, wav)
